# Initial kernel scaffold; baseline (speedup 1.0000x reference)
import jax
import jax.numpy as jnp
from jax.experimental import pallas as pl


def kernel(x, emb, W1, b1, W2, b2):
    raise NotImplementedError("write your pallas kernel here")


# speedup vs baseline: 7.9693x; 7.9693x over previous
import functools

import jax
import jax.numpy as jnp
from jax import lax
from jax.experimental import pallas as pl
from jax.experimental.pallas import tpu as pltpu
from jax.experimental.pallas import tpu_sc as plsc

B = 16384
L = 200
D = 16
NW = 32
BPW = B // NW
CE = 16
CROWS = CE * L
NSTREAM = CROWS // 128
NCHUNK = BPW // CE
IDXROWS = B * L // 128
IDXROWS_PW = IDXROWS // NW


SGROWS = 200
NSG = IDXROWS_PW // SGROWS
NSUB = SGROWS // NSTREAM


def _sc_pool_body(x_hbm, emb_hbm, out_hbm, idx_v, rows_v, pooled_v, sem):
  info = plsc.get_sparse_core_info()
  nc = info.num_cores
  wid = lax.axis_index("s") * nc + lax.axis_index("c")
  inv = jnp.float32(1.0 / L)
  zero = jnp.zeros((16,), jnp.float32)

  def super_body(sg, carry):
    row0 = wid * IDXROWS_PW + sg * SGROWS
    pltpu.sync_copy(x_hbm.at[pl.ds(row0, SGROWS)], idx_v)

    def chunk_body(c, inner_carry):
      copies = [
          pltpu.async_copy(
              emb_hbm.at[idx_v.at[c * NSTREAM + j]],
              rows_v.at[pl.ds(j * 128, 128)], sem)
          for j in range(NSTREAM)
      ]
      for cp in copies:
        cp.wait()
      for e in range(CE):
        def rbody(jj, accs, base=e * L):
          a0, a1, a2, a3 = accs
          o = base + jj * 4
          return (a0 + rows_v[o], a1 + rows_v[o + 1],
                  a2 + rows_v[o + 2], a3 + rows_v[o + 3])
        a0, a1, a2, a3 = lax.fori_loop(0, L // 4, rbody,
                                       (zero, zero, zero, zero))
        pooled_v[e] = ((a0 + a1) + (a2 + a3)) * inv
      pltpu.sync_copy(
          pooled_v,
          out_hbm.at[pl.ds(wid * BPW + sg * (NSUB * CE) + c * CE, CE)])
      return inner_carry

    lax.fori_loop(0, NSUB, chunk_body, 0)
    return carry

  lax.fori_loop(0, NSG, super_body, 0)


@jax.jit
def _sc_pool(x2, emb):
  mesh = plsc.VectorSubcoreMesh(core_axis_name="c", subcore_axis_name="s")
  k = pl.kernel(
      _sc_pool_body,
      out_type=jax.ShapeDtypeStruct((B, D), jnp.float32),
      mesh=mesh,
      scratch_types=[
          pltpu.VMEM((SGROWS, 128), jnp.int32),
          pltpu.VMEM((CROWS, D), jnp.float32),
          pltpu.VMEM((CE, D), jnp.float32),
          pltpu.SemaphoreType.DMA,
      ],
      compiler_params=pltpu.CompilerParams(use_tc_tiling_on_sc=False),
  )
  return k(x2, emb)


def _mlp_body(p_ref, w1_ref, b1_ref, w2_ref, b2_ref, o_ref):
  h = jnp.dot(p_ref[...], w1_ref[...], preferred_element_type=jnp.float32)
  h = jnp.maximum(h + b1_ref[...], 0.0)
  o_ref[...] = (
      jnp.dot(h, w2_ref[...], preferred_element_type=jnp.float32)
      + b2_ref[...])


@jax.jit
def _mlp(pooled, W1, b1, W2, b2):
  bt = 512
  grid = (B // bt,)
  return pl.pallas_call(
      _mlp_body,
      grid=grid,
      in_specs=[
          pl.BlockSpec((bt, D), lambda i: (i, 0)),
          pl.BlockSpec((D, 200), lambda i: (0, 0)),
          pl.BlockSpec((1, 200), lambda i: (0, 0)),
          pl.BlockSpec((200, 128), lambda i: (0, 0)),
          pl.BlockSpec((1, 128), lambda i: (0, 0)),
      ],
      out_specs=pl.BlockSpec((bt, 128), lambda i: (i, 0)),
      out_shape=jax.ShapeDtypeStruct((B, 128), jnp.float32),
  )(pooled, W1, b1, W2, b2)


def kernel(x, emb, W1, b1, W2, b2):
  x2 = x.reshape(IDXROWS, 128)
  pooled = _sc_pool(x2, emb)
  return _mlp(pooled, W1, b1.reshape(1, 200), W2, b2.reshape(1, 128))


# speedup vs baseline: 9.2788x; 1.1643x over previous
import jax
import jax.numpy as jnp
from jax import lax
from jax.experimental import pallas as pl
from jax.experimental.pallas import tpu as pltpu
from jax.experimental.pallas import tpu_sc as plsc

B = 16384
L = 200
D = 16
NW = 32
BPW = B // NW
CE = 16
CROWS = CE * L
NSTREAM = CROWS // 128
NCHUNK = BPW // CE
IDXROWS = B * L // 128
IDXROWS_PW = IDXROWS // NW


def _sc_pool_body(x_hbm, emb_hbm, out_hbm,
                  idx_v0, idx_v1, rows_v0, rows_v1, pooled_v, sem0, sem1):
  info = plsc.get_sparse_core_info()
  nc = info.num_cores
  wid = lax.axis_index("s") * nc + lax.axis_index("c")
  inv = jnp.float32(1.0 / L)
  zero = jnp.zeros((16,), jnp.float32)

  def fire(g, idx_v, rows_v, sem):
    row0 = wid * IDXROWS_PW + g * NSTREAM
    pltpu.sync_copy(x_hbm.at[pl.ds(row0, NSTREAM)], idx_v)
    for j in range(NSTREAM):
      pltpu.async_copy(
          emb_hbm.at[idx_v.at[j]], rows_v.at[pl.ds(j * 128, 128)], sem)

  def drain(rows_v, sem):
    pltpu.make_async_copy(emb_hbm.at[pl.ds(0, CROWS)], rows_v, sem).wait()

  def accumulate(g, rows_v):
    for e in range(CE):
      def rbody(jj, accs, base=e * L):
        o = base + jj * 8
        return tuple(a + rows_v[o + u] for u, a in enumerate(accs))
      a = lax.fori_loop(0, L // 8, rbody, (zero,) * 8)
      s = ((a[0] + a[1]) + (a[2] + a[3])) + ((a[4] + a[5]) + (a[6] + a[7]))
      pooled_v[g * CE + e] = s * inv

  fire(0, idx_v0, rows_v0, sem0)
  fire(1, idx_v1, rows_v1, sem1)

  def loop_body(g0, carry):
    for b, (idx_v, rows_v, sem) in enumerate(
        ((idx_v0, rows_v0, sem0), (idx_v1, rows_v1, sem1))):
      g = g0 * 2 + b
      drain(rows_v, sem)
      accumulate(g, rows_v)

      @pl.when(g < NCHUNK - 2)
      def _():
        fire(g + 2, idx_v, rows_v, sem)
    return carry

  lax.fori_loop(0, NCHUNK // 2, loop_body, 0)
  pltpu.sync_copy(pooled_v, out_hbm.at[pl.ds(wid * BPW, BPW)])


@jax.jit
def _sc_pool(x2, emb):
  mesh = plsc.VectorSubcoreMesh(core_axis_name="c", subcore_axis_name="s")
  k = pl.kernel(
      _sc_pool_body,
      out_type=jax.ShapeDtypeStruct((B, D), jnp.float32),
      mesh=mesh,
      scratch_types=[
          pltpu.VMEM((NSTREAM, 128), jnp.int32),
          pltpu.VMEM((NSTREAM, 128), jnp.int32),
          pltpu.VMEM((CROWS, D), jnp.float32),
          pltpu.VMEM((CROWS, D), jnp.float32),
          pltpu.VMEM((BPW, D), jnp.float32),
          pltpu.SemaphoreType.DMA,
          pltpu.SemaphoreType.DMA,
      ],
      compiler_params=pltpu.CompilerParams(use_tc_tiling_on_sc=False),
  )
  return k(x2, emb)


def _mlp_body(p_ref, w1_ref, b1_ref, w2_ref, b2_ref, o_ref):
  h = jnp.dot(p_ref[...], w1_ref[...], preferred_element_type=jnp.float32)
  h = jnp.maximum(h + b1_ref[...], 0.0)
  o_ref[...] = (
      jnp.dot(h, w2_ref[...], preferred_element_type=jnp.float32)
      + b2_ref[...])


@jax.jit
def _mlp(pooled, W1, b1, W2, b2):
  bt = 512
  grid = (B // bt,)
  return pl.pallas_call(
      _mlp_body,
      grid=grid,
      in_specs=[
          pl.BlockSpec((bt, D), lambda i: (i, 0)),
          pl.BlockSpec((D, 200), lambda i: (0, 0)),
          pl.BlockSpec((1, 200), lambda i: (0, 0)),
          pl.BlockSpec((200, 128), lambda i: (0, 0)),
          pl.BlockSpec((1, 128), lambda i: (0, 0)),
      ],
      out_specs=pl.BlockSpec((bt, 128), lambda i: (i, 0)),
      out_shape=jax.ShapeDtypeStruct((B, 128), jnp.float32),
  )(pooled, W1, b1, W2, b2)


def kernel(x, emb, W1, b1, W2, b2):
  x2 = x.reshape(IDXROWS, 128)
  pooled = _sc_pool(x2, emb)
  return _mlp(pooled, W1, b1.reshape(1, 200), W2, b2.reshape(1, 128))


# speedup vs baseline: 21.3442x; 2.3003x over previous
import jax
import jax.numpy as jnp
from jax import lax
from jax.experimental import pallas as pl
from jax.experimental.pallas import tpu as pltpu
from jax.experimental.pallas import tpu_sc as plsc

B = 16384
L = 200
D = 16
V = 1000000
NW = 32
BPW = B // NW
CE = 16
CROWS = CE * L
NSTREAM = CROWS // 128
NCHUNK = BPW // CE
IDXROWS = B * L // 128
IDXROWS_PW = IDXROWS // NW

SH_R = 1024
SH_FULL = 976
SH_TRIPS = 32
OUT_ROWS = V * D // 128


def _shuffle_body(embt_hbm, tail_hbm, out_hbm,
                  st0, st1, ob0, ob1, tb, isem0, isem1, osem0, osem1):
  info = plsc.get_sparse_core_info()
  nc = info.num_cores
  wid = lax.axis_index("s") * nc + lax.axis_index("c")
  stride16 = lax.iota(jnp.int32, 16) * 16

  def fire_in(i, st, isem):
    c = wid + 32 * i
    pltpu.async_copy(embt_hbm.at[:, pl.ds(c * SH_R, SH_R)], st, isem)

  def shuffle(st, ob, nrows):
    @plsc.parallel_loop(0, nrows // 16, unroll=2)
    def _(t):
      rr0 = t * 16
      base = stride16 + rr0 * 16
      for d in range(D):
        val = st[d, pl.ds(rr0, 16)]
        plsc.store_scatter(ob, [base + d], val)

  fire_in(0, st0, isem0)
  fire_in(1, st1, isem1)

  def loop_body(i0, carry):
    for bsel, (st, ob, isem, osem) in enumerate(
        ((st0, ob0, isem0, osem0), (st1, ob1, isem1, osem1))):
      i = i0 * 2 + bsel
      c = wid + 32 * i

      @pl.when(c < SH_FULL)
      def _():
        pltpu.make_async_copy(
            embt_hbm.at[:, pl.ds(0, SH_R)], st, isem).wait()

        @pl.when(i >= 2)
        def _():
          pltpu.make_async_copy(
              ob, out_hbm.at[pl.ds(0, SH_R * D)], osem).wait()
        shuffle(st, ob, SH_R)

        @pl.when(c + 64 < SH_FULL)
        def _():
          fire_in(i + 2, st, isem)
        pltpu.async_copy(ob, out_hbm.at[pl.ds(c * (SH_R * D), SH_R * D)],
                         osem)
    return carry

  lax.fori_loop(0, SH_TRIPS // 2, loop_body, 0)
  pltpu.make_async_copy(ob0, out_hbm.at[pl.ds(0, SH_R * D)], osem0).wait()
  pltpu.make_async_copy(ob1, out_hbm.at[pl.ds(0, SH_R * D)], osem1).wait()

  @pl.when(wid == 16)
  def _():
    pltpu.async_copy(
        embt_hbm.at[:, pl.ds(SH_FULL * SH_R, 512)],
        st0.at[:, pl.ds(0, 512)], isem0)
    pltpu.make_async_copy(
        embt_hbm.at[:, pl.ds(0, 512)], st0.at[:, pl.ds(0, 512)],
        isem0).wait()
    shuffle(st0, ob0, 512)
    pltpu.sync_copy(ob0.at[pl.ds(0, 512 * D)],
                    out_hbm.at[pl.ds(SH_FULL * (SH_R * D), 512 * D)])

  @pl.when(wid == 17)
  def _():
    pltpu.sync_copy(tail_hbm, tb)
    pltpu.sync_copy(tb, out_hbm.at[pl.ds(V * D - 1024, 1024)])


@jax.jit
def _sc_shuffle(embt, tail):
  mesh = plsc.VectorSubcoreMesh(core_axis_name="c", subcore_axis_name="s")
  k = pl.kernel(
      _shuffle_body,
      out_type=jax.ShapeDtypeStruct((V * D,), jnp.float32),
      mesh=mesh,
      scratch_types=[
          pltpu.VMEM((D, SH_R), jnp.float32),
          pltpu.VMEM((D, SH_R), jnp.float32),
          pltpu.VMEM((SH_R * D,), jnp.float32),
          pltpu.VMEM((SH_R * D,), jnp.float32),
          pltpu.VMEM((1024,), jnp.float32),
          pltpu.SemaphoreType.DMA,
          pltpu.SemaphoreType.DMA,
          pltpu.SemaphoreType.DMA,
          pltpu.SemaphoreType.DMA,
      ],
      compiler_params=pltpu.CompilerParams(
          use_tc_tiling_on_sc=True, needs_layout_passes=False),
  )
  return k(embt, tail)


def _sc_pool_body(x_hbm, emb_hbm, out_hbm,
                  idx_v0, idx_v1, rows_v0, rows_v1, pooled_v,
                  sem0, sem1, isem0, isem1):
  info = plsc.get_sparse_core_info()
  nc = info.num_cores
  wid = lax.axis_index("s") * nc + lax.axis_index("c")
  inv = jnp.float32(1.0 / L)
  zero = jnp.zeros((16,), jnp.float32)

  def fire_idx(g, idx_v, isem):
    row0 = wid * IDXROWS_PW + g * NSTREAM
    pltpu.async_copy(x_hbm.at[pl.ds(row0, NSTREAM)], idx_v, isem)

  def fire(idx_v, rows_v, sem, isem):
    pltpu.make_async_copy(x_hbm.at[pl.ds(0, NSTREAM)], idx_v, isem).wait()
    for j in range(NSTREAM):
      pltpu.async_copy(
          emb_hbm.at[idx_v.at[j]], rows_v.at[pl.ds(j * 128, 128)], sem)

  def drain(rows_v, sem):
    pltpu.make_async_copy(emb_hbm.at[pl.ds(0, CROWS)], rows_v, sem).wait()

  def accumulate(g, rows_v):
    for e in range(CE):
      @plsc.parallel_loop(0, L // 8, unroll=2, carry=(zero,) * 8)
      def a(jj, accs, base=e * L):
        o = base + jj * 8
        return tuple(x + rows_v[o + u] for u, x in enumerate(accs))
      s = ((a[0] + a[1]) + (a[2] + a[3])) + ((a[4] + a[5]) + (a[6] + a[7]))
      pooled_v[g * CE + e] = s * inv

  fire_idx(0, idx_v0, isem0)
  fire_idx(1, idx_v1, isem1)
  fire(idx_v0, rows_v0, sem0, isem0)
  fire(idx_v1, rows_v1, sem1, isem1)

  def loop_body(g0, carry):
    for bsel, (idx_v, rows_v, sem, isem) in enumerate(
        ((idx_v0, rows_v0, sem0, isem0), (idx_v1, rows_v1, sem1, isem1))):
      g = g0 * 2 + bsel
      drain(rows_v, sem)

      @pl.when(g < NCHUNK - 2)
      def _():
        fire_idx(g + 2, idx_v, isem)
      accumulate(g, rows_v)

      @pl.when(g < NCHUNK - 2)
      def _():
        fire(idx_v, rows_v, sem, isem)
    return carry

  lax.fori_loop(0, NCHUNK // 2, loop_body, 0)
  pltpu.sync_copy(pooled_v, out_hbm.at[pl.ds(wid * BPW, BPW)])


@jax.jit
def _sc_pool(x2, emb_lin):
  mesh = plsc.VectorSubcoreMesh(core_axis_name="c", subcore_axis_name="s")
  k = pl.kernel(
      _sc_pool_body,
      out_type=jax.ShapeDtypeStruct((B, D), jnp.float32),
      mesh=mesh,
      scratch_types=[
          pltpu.VMEM((NSTREAM, 128), jnp.int32),
          pltpu.VMEM((NSTREAM, 128), jnp.int32),
          pltpu.VMEM((CROWS, D), jnp.float32),
          pltpu.VMEM((CROWS, D), jnp.float32),
          pltpu.VMEM((BPW, D), jnp.float32),
          pltpu.SemaphoreType.DMA,
          pltpu.SemaphoreType.DMA,
          pltpu.SemaphoreType.DMA,
          pltpu.SemaphoreType.DMA,
      ],
      compiler_params=pltpu.CompilerParams(use_tc_tiling_on_sc=False),
  )
  return k(x2, emb_lin)


def _mlp_body(p_ref, w1_ref, b1_ref, w2_ref, b2_ref, o_ref):
  h = jnp.dot(p_ref[...], w1_ref[...], preferred_element_type=jnp.float32)
  h = jnp.maximum(h + b1_ref[...], 0.0)
  o_ref[...] = (
      jnp.dot(h, w2_ref[...], preferred_element_type=jnp.float32)
      + b2_ref[...])


@jax.jit
def _mlp(pooled, W1, b1, W2, b2):
  bt = 512
  grid = (B // bt,)
  return pl.pallas_call(
      _mlp_body,
      grid=grid,
      in_specs=[
          pl.BlockSpec((bt, D), lambda i: (i, 0)),
          pl.BlockSpec((D, 200), lambda i: (0, 0)),
          pl.BlockSpec((1, 200), lambda i: (0, 0)),
          pl.BlockSpec((200, 128), lambda i: (0, 0)),
          pl.BlockSpec((1, 128), lambda i: (0, 0)),
      ],
      out_specs=pl.BlockSpec((bt, 128), lambda i: (i, 0)),
      out_shape=jax.ShapeDtypeStruct((B, 128), jnp.float32),
  )(pooled, W1, b1, W2, b2)


def kernel(x, emb, W1, b1, W2, b2):
  x2 = x.reshape(IDXROWS, 128)
  embt = emb.T
  tail = emb[V - 64:].reshape(1024)
  emb_lin = _sc_shuffle(embt, tail).reshape(V, D)
  pooled = _sc_pool(x2, emb_lin)
  return _mlp(pooled, W1, b1.reshape(1, 200), W2, b2.reshape(1, 128))
